# baseline (speedup 1.0000x reference)
import functools

import jax
import jax.numpy as jnp
from jax import lax
from jax.experimental import pallas as pl
from jax.experimental.pallas import tpu as pltpu
from jax.experimental.pallas import tpu_sc as plsc

PRE = 4000
PRE_PAD = 4096
POST = 1000
OUT_PAD = 1024
NB = 32
B = 128
THR = 0.7
PADV = -1e6


def _nms_select_body(bT_r, s_r, out_r, keep_r, hit_r, data_r):
    f32 = jnp.float32
    N = PRE_PAD
    sub_bb = jax.lax.broadcasted_iota(jnp.int32, (B, B), 0)
    lane_bb = jax.lax.broadcasted_iota(jnp.int32, (B, B), 1)
    ident = (sub_bb == lane_bb).astype(f32)
    tri_strict = (sub_bb < lane_bb).astype(f32)
    tri_incl = (sub_bb <= lane_bb).astype(f32)
    ones_col = jnp.ones((B, 1), f32)

    def t_row(v):
        return jax.lax.dot_general(ident, v, (((1,), (1,)), ((), ())),
                                   preferred_element_type=f32)

    def mm(a, b):
        return jax.lax.dot_general(a, b, (((1,), (0,)), ((), ())),
                                   preferred_element_type=f32)

    bf16 = jnp.bfloat16
    lane_n = jax.lax.broadcasted_iota(jnp.int32, (1, N), 1)
    real = lane_n < PRE
    x1 = jnp.where(real, bT_r[0:1, :], PADV)
    y1 = jnp.where(real, bT_r[1:2, :], PADV)
    x2 = jnp.where(real, bT_r[2:3, :], PADV)
    y2 = jnp.where(real, bT_r[3:4, :], PADV)
    s_row = s_r[...]
    areas = jnp.maximum(x2 - x1, 0.0) * jnp.maximum(y2 - y1, 0.0)
    carea_all = (THR / (1.0 + THR)) * areas
    tri_strict_bf = tri_strict.astype(bf16)

    for a in range(NB):
        s0 = a * B
        rx1 = t_row(x1[:, s0:s0 + B])
        ry1 = t_row(y1[:, s0:s0 + B])
        rx2 = t_row(x2[:, s0:s0 + B])
        ry2 = t_row(y2[:, s0:s0 + B])
        rcarea = t_row(carea_all[:, s0:s0 + B])

        scol = t_row(s_row[:, s0:s0 + B])
        data_r[pl.ds(s0, B), :] = jnp.concatenate(
            [jnp.zeros((B, 1), f32), scol, rx1, ry1, rx2, ry2,
             jnp.zeros((B, 2), f32)], axis=1)

        xx1 = jnp.maximum(rx1, x1[:, s0:])
        yy1 = jnp.maximum(ry1, y1[:, s0:])
        xx2 = jnp.minimum(rx2, x2[:, s0:])
        yy2 = jnp.minimum(ry2, y2[:, s0:])
        inter = jnp.maximum(xx2 - xx1, 0.0) * jnp.maximum(yy2 - yy1, 0.0)
        thresh = rcarea + carea_all[:, s0:]
        hit_r[pl.ds(a * B, B), pl.ds(s0, N - s0)] = jnp.where(
            inter > thresh, 1.0, 0.0).astype(bf16)

    keep_r[...] = jnp.ones((1, N), f32)

    def mm_bf(k, h):
        return jax.lax.dot_general(k.astype(bf16), h, (((1,), (0,)), ((), ())),
                                   preferred_element_type=f32)

    for a in range(NB):
        s0 = a * B
        mh = hit_r[pl.ds(a * B, B), pl.ds(s0, B)] * tri_strict_bf
        k0 = keep_r[:, s0:s0 + B]

        def w_body(c):
            k, _ = c
            k2 = jnp.where(mm_bf(k, mh) > 0.0, 0.0, k0)
            return (k2, jnp.any(k2 != k))

        k1 = jnp.where(mm_bf(k0, mh) > 0.0, 0.0, k0)
        kfin, _ = jax.lax.while_loop(lambda c: c[1], w_body,
                                     (k1, jnp.any(k1 != k0)))
        keep_r[:, s0:s0 + B] = kfin

        if a + 1 < NB:
            htail = hit_r[pl.ds(a * B, B), pl.ds(s0 + B, N - s0 - B)]
            cnt = mm_bf(kfin, htail)
            tail = keep_r[:, s0 + B:]
            keep_r[:, s0 + B:] = jnp.where(cnt > 0.0, 0.0, tail)

    p_sub = jax.lax.broadcasted_iota(jnp.int32, (OUT_PAD, 1), 0).astype(f32)
    lane_b = jax.lax.broadcasted_iota(jnp.int32, (1, B), 1)
    acc = jnp.zeros((OUT_PAD, 8), f32)
    off = jnp.zeros((1, 1), f32)
    for a in range(NB):
        s0 = a * B
        kb = keep_r[:, s0:s0 + B]
        if s0 + B > PRE:
            kb = jnp.where(lane_b + s0 < PRE, kb, 0.0)
        incl = mm(kb, tri_incl)
        excl = incl - kb + off
        slots = jnp.where(kb > 0.5, excl, -1.0)
        onehot = jnp.where(p_sub == slots, 1.0, 0.0)
        acc = acc + mm(onehot, data_r[pl.ds(s0, B), :])
        off = off + mm(kb, ones_col)
    padmask = jnp.where(p_sub >= off, 1.0, 0.0)
    out_r[...] = acc + padmask * data_r[PRE - 1:PRE, :]


def _nms_select(ballT, spad):
    return pl.pallas_call(
        _nms_select_body,
        out_shape=jax.ShapeDtypeStruct((OUT_PAD, 8), jnp.float32),
        in_specs=[
            pl.BlockSpec((4, PRE_PAD), lambda: (0, 0)),
            pl.BlockSpec((1, PRE_PAD), lambda: (0, 0)),
        ],
        out_specs=pl.BlockSpec((OUT_PAD, 8), lambda: (0, 0)),
        scratch_shapes=[
            pltpu.VMEM((1, PRE_PAD), jnp.float32),
            pltpu.VMEM((PRE_PAD, PRE_PAD), jnp.bfloat16),
            pltpu.VMEM((PRE_PAD, 8), jnp.float32),
        ],
    )(ballT, spad)


NBOX = 20000
_SC_INFO = plsc.get_sparse_core_info()
NW = _SC_INFO.num_cores * _SC_INFO.num_subcores
BPW = PRE_PAD // NW
L = 16


HALF = NBOX * 2


def _sc_gather(blo, bhi, order, s):
    i32 = jnp.int32
    f32 = jnp.float32
    mesh = plsc.VectorSubcoreMesh(core_axis_name="c", subcore_axis_name="s")

    @functools.partial(
        pl.kernel, mesh=mesh,
        compiler_params=pltpu.CompilerParams(needs_layout_passes=False),
        out_type=[jax.ShapeDtypeStruct((4 * PRE_PAD,), f32),
                  jax.ShapeDtypeStruct((PRE_PAD,), f32)],
        scratch_types=[
            pltpu.VMEM((HALF,), f32),
            pltpu.VMEM((HALF,), f32),
            pltpu.VMEM((BPW,), i32),
            pltpu.VMEM((4, BPW), f32),
            pltpu.VMEM((BPW,), f32),
            pltpu.SemaphoreType.DMA,
        ],
    )
    def k(blo_hbm, bhi_hbm, order_hbm, s_hbm, outb_hbm, outs_hbm,
          tlo_v, thi_v, idx_v, rows_v, sv_v, sem):
        wid = lax.axis_index("s") * _SC_INFO.num_cores + lax.axis_index("c")
        base = wid * BPW
        is_last = wid == NW - 1
        n_last = PRE - (NW - 1) * BPW

        pltpu.sync_copy(blo_hbm, tlo_v)
        pltpu.sync_copy(bhi_hbm, thi_v)

        @pl.when(jnp.logical_not(is_last))
        def _():
            pltpu.sync_copy(order_hbm.at[pl.ds(base, BPW)], idx_v)
            pltpu.sync_copy(s_hbm.at[pl.ds(base, BPW)], sv_v)

        @pl.when(is_last)
        def _():
            for v in range(BPW // L):
                idx_v[pl.ds(v * L, L)] = jnp.zeros((L,), i32)
                sv_v[pl.ds(v * L, L)] = jnp.zeros((L,), f32)
            pltpu.sync_copy(order_hbm.at[pl.ds((NW - 1) * BPW, n_last)],
                            idx_v.at[pl.ds(0, n_last)])
            pltpu.sync_copy(s_hbm.at[pl.ds((NW - 1) * BPW, n_last)],
                            sv_v.at[pl.ds(0, n_last)])

        half_vec = jnp.full((L,), HALF, i32)
        maxlo = jnp.full((L,), HALF - 1, i32)
        zero_vec = jnp.zeros((L,), i32)
        for v in range(BPW // L):
            iv4 = idx_v[pl.ds(v * L, L)] * 4
            for c in range(4):
                gidx = iv4 + c
                inlo = gidx < half_vec
                glo = plsc.load_gather(tlo_v, [jnp.minimum(gidx, maxlo)])
                ghi = plsc.load_gather(thi_v, [jnp.maximum(gidx - half_vec, zero_vec)])
                rows_v[c, pl.ds(v * L, L)] = jnp.where(inlo, glo, ghi)

        for c in range(4):
            pltpu.sync_copy(rows_v.at[c],
                            outb_hbm.at[pl.ds(c * PRE_PAD + base, BPW)])
        pltpu.sync_copy(sv_v, outs_hbm.at[pl.ds(base, BPW)])

    return k(blo, bhi, order, s)


def kernel(boxes, scores, pre_nms_top_n, post_nms_top_n):
    f32 = jnp.float32
    s, order = jax.lax.top_k(scores, PRE)
    flat = boxes.astype(f32).reshape(4 * NBOX)
    ballT_flat, spad_flat = _sc_gather(flat[:HALF], flat[HALF:],
                                       order.astype(jnp.int32), s.astype(f32))
    ballT = ballT_flat.reshape(4, PRE_PAD)
    spad = spad_flat.reshape(1, PRE_PAD)
    out = _nms_select(ballT, spad)
    return out[:POST, :6][None, :, :]


# speedup vs baseline: 1.6086x; 1.6086x over previous
import jax
import jax.numpy as jnp
from jax.experimental import pallas as pl
from jax.experimental.pallas import tpu as pltpu

PRE = 4000
PRE_PAD = 4096
POST = 1000
OUT_PAD = 1024
NB = 32
B = 128
THR = 0.7
PADV = -1e6


def _nms_select_body(order_r, boxes_r, s_r, out_r, keep_r, hit_r, data_r, bx_r):
    f32 = jnp.float32
    N = PRE_PAD
    sub_bb = jax.lax.broadcasted_iota(jnp.int32, (B, B), 0)
    lane_bb = jax.lax.broadcasted_iota(jnp.int32, (B, B), 1)
    ident = (sub_bb == lane_bb).astype(f32)
    tri_strict = (sub_bb < lane_bb).astype(f32)
    tri_incl = (sub_bb <= lane_bb).astype(f32)
    ones_col = jnp.ones((B, 1), f32)

    def t_row(v):
        return jax.lax.dot_general(ident, v, (((1,), (1,)), ((), ())),
                                   preferred_element_type=f32)

    def mm(a, b):
        return jax.lax.dot_general(a, b, (((1,), (0,)), ((), ())),
                                   preferred_element_type=f32)

    bf16 = jnp.bfloat16

    for i in range(PRE):
        bx_r[pl.ds(i, 1), :] = boxes_r[pl.ds(order_r[i], 1), :]

    ball = jnp.concatenate(
        [bx_r[...], jnp.full((PRE_PAD - PRE, 4), PADV, f32)], axis=0)
    bT = jnp.transpose(ball)
    x1 = bT[0:1, :]
    y1 = bT[1:2, :]
    x2 = bT[2:3, :]
    y2 = bT[3:4, :]
    s_row = s_r[...]
    areas = jnp.maximum(x2 - x1, 0.0) * jnp.maximum(y2 - y1, 0.0)
    carea_all = (THR / (1.0 + THR)) * areas
    tri_strict_bf = tri_strict.astype(bf16)

    for a in range(NB):
        s0 = a * B
        rx1 = t_row(x1[:, s0:s0 + B])
        ry1 = t_row(y1[:, s0:s0 + B])
        rx2 = t_row(x2[:, s0:s0 + B])
        ry2 = t_row(y2[:, s0:s0 + B])
        rcarea = t_row(carea_all[:, s0:s0 + B])

        scol = t_row(s_row[:, s0:s0 + B])
        data_r[pl.ds(s0, B), :] = jnp.concatenate(
            [jnp.zeros((B, 1), f32), scol, rx1, ry1, rx2, ry2,
             jnp.zeros((B, 2), f32)], axis=1)

        xx1 = jnp.maximum(rx1, x1[:, s0:])
        yy1 = jnp.maximum(ry1, y1[:, s0:])
        xx2 = jnp.minimum(rx2, x2[:, s0:])
        yy2 = jnp.minimum(ry2, y2[:, s0:])
        inter = jnp.maximum(xx2 - xx1, 0.0) * jnp.maximum(yy2 - yy1, 0.0)
        thresh = rcarea + carea_all[:, s0:]
        hit_r[pl.ds(a * B, B), pl.ds(s0, N - s0)] = jnp.where(
            inter > thresh, 1.0, 0.0).astype(bf16)

    keep_r[...] = jnp.ones((1, N), f32)

    def mm_bf(k, h):
        return jax.lax.dot_general(k.astype(bf16), h, (((1,), (0,)), ((), ())),
                                   preferred_element_type=f32)

    for a in range(NB):
        s0 = a * B
        mh = hit_r[pl.ds(a * B, B), pl.ds(s0, B)] * tri_strict_bf
        k0 = keep_r[:, s0:s0 + B]

        def w_body(c):
            k, _ = c
            k2 = jnp.where(mm_bf(k, mh) > 0.0, 0.0, k0)
            return (k2, jnp.any(k2 != k))

        k1 = jnp.where(mm_bf(k0, mh) > 0.0, 0.0, k0)
        kfin, _ = jax.lax.while_loop(lambda c: c[1], w_body,
                                     (k1, jnp.any(k1 != k0)))
        keep_r[:, s0:s0 + B] = kfin

        if a + 1 < NB:
            htail = hit_r[pl.ds(a * B, B), pl.ds(s0 + B, N - s0 - B)]
            cnt = mm_bf(kfin, htail)
            tail = keep_r[:, s0 + B:]
            keep_r[:, s0 + B:] = jnp.where(cnt > 0.0, 0.0, tail)

    p_sub = jax.lax.broadcasted_iota(jnp.int32, (OUT_PAD, 1), 0).astype(f32)
    lane_b = jax.lax.broadcasted_iota(jnp.int32, (1, B), 1)
    acc = jnp.zeros((OUT_PAD, 8), f32)
    off = jnp.zeros((1, 1), f32)
    for a in range(NB):
        s0 = a * B
        kb = keep_r[:, s0:s0 + B]
        if s0 + B > PRE:
            kb = jnp.where(lane_b + s0 < PRE, kb, 0.0)
        incl = mm(kb, tri_incl)
        excl = incl - kb + off
        slots = jnp.where(kb > 0.5, excl, -1.0)
        onehot = jnp.where(p_sub == slots, 1.0, 0.0)
        acc = acc + mm(onehot, data_r[pl.ds(s0, B), :])
        off = off + mm(kb, ones_col)
    padmask = jnp.where(p_sub >= off, 1.0, 0.0)
    out_r[...] = acc + padmask * data_r[PRE - 1:PRE, :]


def _nms_select(order, boxes, spad):
    return pl.pallas_call(
        _nms_select_body,
        out_shape=jax.ShapeDtypeStruct((OUT_PAD, 8), jnp.float32),
        in_specs=[
            pl.BlockSpec(memory_space=pltpu.SMEM),
            pl.BlockSpec((20000, 4), lambda: (0, 0)),
            pl.BlockSpec((1, PRE_PAD), lambda: (0, 0)),
        ],
        out_specs=pl.BlockSpec((OUT_PAD, 8), lambda: (0, 0)),
        scratch_shapes=[
            pltpu.VMEM((1, PRE_PAD), jnp.float32),
            pltpu.VMEM((PRE_PAD, PRE_PAD), jnp.bfloat16),
            pltpu.VMEM((PRE_PAD, 8), jnp.float32),
            pltpu.VMEM((PRE, 4), jnp.float32),
        ],
    )(order, boxes, spad)


def kernel(boxes, scores, pre_nms_top_n, post_nms_top_n):
    f32 = jnp.float32
    s, order = jax.lax.top_k(scores, PRE)
    spad = jnp.concatenate(
        [s.astype(f32), jnp.zeros((PRE_PAD - PRE,), f32)]).reshape(1, PRE_PAD)
    out = _nms_select(order.astype(jnp.int32), boxes.astype(f32), spad)
    return out[:POST, :6][None, :, :]


# speedup vs baseline: 1.6229x; 1.0089x over previous
import jax
import jax.numpy as jnp
from jax.experimental import pallas as pl
from jax.experimental.pallas import tpu as pltpu

PRE = 4000
PRE_PAD = 4096
POST = 1000
OUT_PAD = 1024
NB = 32
B = 128
THR = 0.7
PADV = -1e6


def _nms_select_body(order_r, boxes_r, s_r, out_r, keep_r, hit_r, data_r, bx_r):
    f32 = jnp.float32
    N = PRE_PAD
    sub_bb = jax.lax.broadcasted_iota(jnp.int32, (B, B), 0)
    lane_bb = jax.lax.broadcasted_iota(jnp.int32, (B, B), 1)
    ident = (sub_bb == lane_bb).astype(f32)
    tri_strict = (sub_bb < lane_bb).astype(f32)
    tri_incl = (sub_bb <= lane_bb).astype(f32)
    ones_col = jnp.ones((B, 1), f32)

    def t_row(v):
        return jax.lax.dot_general(ident, v, (((1,), (1,)), ((), ())),
                                   preferred_element_type=f32)

    def mm(a, b):
        return jax.lax.dot_general(a, b, (((1,), (0,)), ((), ())),
                                   preferred_element_type=f32)

    bf16 = jnp.bfloat16

    for i in range(PRE):
        bx_r[pl.ds(i, 1), :] = boxes_r[pl.ds(order_r[i], 1), :]

    ball = jnp.concatenate(
        [bx_r[...], jnp.full((PRE_PAD - PRE, 4), PADV, f32)], axis=0)
    bT = jnp.transpose(ball)
    x1 = bT[0:1, :]
    y1 = bT[1:2, :]
    x2 = bT[2:3, :]
    y2 = bT[3:4, :]
    s_row = jnp.concatenate(
        [s_r[...], jnp.zeros((1, PRE_PAD - PRE), f32)], axis=1)
    areas = jnp.maximum(x2 - x1, 0.0) * jnp.maximum(y2 - y1, 0.0)
    carea_all = (THR / (1.0 + THR)) * areas
    tri_strict_bf = tri_strict.astype(bf16)

    for a in range(NB):
        s0 = a * B
        rx1 = t_row(x1[:, s0:s0 + B])
        ry1 = t_row(y1[:, s0:s0 + B])
        rx2 = t_row(x2[:, s0:s0 + B])
        ry2 = t_row(y2[:, s0:s0 + B])
        rcarea = t_row(carea_all[:, s0:s0 + B])

        scol = t_row(s_row[:, s0:s0 + B])
        data_r[pl.ds(s0, B), :] = jnp.concatenate(
            [jnp.zeros((B, 1), f32), scol, rx1, ry1, rx2, ry2,
             jnp.zeros((B, 2), f32)], axis=1)

        xx1 = jnp.maximum(rx1, x1[:, s0:])
        yy1 = jnp.maximum(ry1, y1[:, s0:])
        xx2 = jnp.minimum(rx2, x2[:, s0:])
        yy2 = jnp.minimum(ry2, y2[:, s0:])
        inter = jnp.maximum(xx2 - xx1, 0.0) * jnp.maximum(yy2 - yy1, 0.0)
        thresh = rcarea + carea_all[:, s0:]
        hit_r[pl.ds(a * B, B), pl.ds(s0, N - s0)] = jnp.where(
            inter > thresh, 1.0, 0.0).astype(bf16)

    keep_r[...] = jnp.ones((1, N), f32)

    def mm_bf(k, h):
        return jax.lax.dot_general(k.astype(bf16), h, (((1,), (0,)), ((), ())),
                                   preferred_element_type=f32)

    for a in range(NB):
        s0 = a * B
        mh = hit_r[pl.ds(a * B, B), pl.ds(s0, B)] * tri_strict_bf
        k0 = keep_r[:, s0:s0 + B]

        def w_body(c):
            k, _ = c
            k2 = jnp.where(mm_bf(k, mh) > 0.0, 0.0, k0)
            return (k2, jnp.any(k2 != k))

        k1 = jnp.where(mm_bf(k0, mh) > 0.0, 0.0, k0)
        kfin, _ = jax.lax.while_loop(lambda c: c[1], w_body,
                                     (k1, jnp.any(k1 != k0)))
        keep_r[:, s0:s0 + B] = kfin

        if a + 1 < NB:
            htail = hit_r[pl.ds(a * B, B), pl.ds(s0 + B, N - s0 - B)]
            cnt = mm_bf(kfin, htail)
            tail = keep_r[:, s0 + B:]
            keep_r[:, s0 + B:] = jnp.where(cnt > 0.0, 0.0, tail)

    p_sub = jax.lax.broadcasted_iota(jnp.int32, (OUT_PAD, 1), 0).astype(f32)
    lane_b = jax.lax.broadcasted_iota(jnp.int32, (1, B), 1)
    acc = jnp.zeros((OUT_PAD, 8), f32)
    off = jnp.zeros((1, 1), f32)
    for a in range(NB):
        s0 = a * B
        kb = keep_r[:, s0:s0 + B]
        if s0 + B > PRE:
            kb = jnp.where(lane_b + s0 < PRE, kb, 0.0)
        incl = mm(kb, tri_incl)
        excl = incl - kb + off
        slots = jnp.where(kb > 0.5, excl, -1.0)
        onehot = jnp.where(p_sub == slots, 1.0, 0.0)
        acc = acc + mm(onehot, data_r[pl.ds(s0, B), :])
        off = off + mm(kb, ones_col)
    padmask = jnp.where(p_sub >= off, 1.0, 0.0)
    out_r[...] = acc + padmask * data_r[PRE - 1:PRE, :]


def _nms_select(order, boxes, spad):
    return pl.pallas_call(
        _nms_select_body,
        out_shape=jax.ShapeDtypeStruct((OUT_PAD, 8), jnp.float32),
        in_specs=[
            pl.BlockSpec(memory_space=pltpu.SMEM),
            pl.BlockSpec((20000, 4), lambda: (0, 0)),
            pl.BlockSpec((1, PRE), lambda: (0, 0)),
        ],
        out_specs=pl.BlockSpec((OUT_PAD, 8), lambda: (0, 0)),
        scratch_shapes=[
            pltpu.VMEM((1, PRE_PAD), jnp.float32),
            pltpu.VMEM((PRE_PAD, PRE_PAD), jnp.bfloat16),
            pltpu.VMEM((PRE_PAD, 8), jnp.float32),
            pltpu.VMEM((PRE, 4), jnp.float32),
        ],
    )(order, boxes, spad)


def kernel(boxes, scores, pre_nms_top_n, post_nms_top_n):
    f32 = jnp.float32
    s, order = jax.lax.top_k(scores, PRE)
    out = _nms_select(order.astype(jnp.int32), boxes.astype(f32),
                      s.astype(f32).reshape(1, PRE))
    return out[:POST, :6][None, :, :]
